# baseline (speedup 1.0000x reference)
import jax
import jax.numpy as jnp
from jax.experimental import pallas as pl
from jax.experimental.pallas import tpu as pltpu


def _gcn_kernel(x_ref, s_ref, w1_ref, b1_ref, w2_ref, b2_ref, wp_ref,
                bp_ref, o_ref, sbf_ref, t2_ref):
    b = pl.program_id(0)
    i = jax.lax.rem(b, 2)
    j = 1 - i

    sb = s_ref[0].astype(jnp.bfloat16)
    sbf_ref[i] = sb
    t1 = jnp.dot(x_ref[0], w1_ref[...], preferred_element_type=jnp.float32)
    h1 = jnp.dot(sb, t1.astype(jnp.bfloat16),
                 preferred_element_type=jnp.float32) + b1_ref[...]
    h1 = jnp.maximum(h1, 0.0)
    t2_ref[i] = jnp.dot(h1, w2_ref[...], preferred_element_type=jnp.float32)

    sp = sbf_ref[j]
    t2 = t2_ref[j]
    h2 = jnp.dot(sp, t2.astype(jnp.bfloat16),
                 preferred_element_type=jnp.float32) + b2_ref[...]
    h2 = jnp.maximum(h2, 0.0)
    mx = jnp.max(h2, axis=0, keepdims=True)
    sm = jnp.sum(h2, axis=0, keepdims=True)
    cat = jnp.concatenate([mx, sm], axis=1)
    o_ref[0] = jnp.dot(cat, wp_ref[...],
                       preferred_element_type=jnp.float32) + bp_ref[...]


def kernel(x, support, W1, b1, W2, b2, Wp, bp):
    B, N, D_IN = x.shape
    H1 = W1.shape[1]
    H2 = W2.shape[1]
    OUT = Wp.shape[1]

    b1_2d = b1.reshape(1, H1)
    b2_2d = b2.reshape(1, H2)
    bp_2d = bp.reshape(1, OUT)

    out = pl.pallas_call(
        _gcn_kernel,
        grid=(B + 1,),
        in_specs=[
            pl.BlockSpec((1, N, D_IN),
                         lambda b: (jnp.minimum(b, B - 1), 0, 0)),
            pl.BlockSpec((1, N, N),
                         lambda b: (jnp.minimum(b, B - 1), 0, 0)),
            pl.BlockSpec((D_IN, H1), lambda b: (0, 0)),
            pl.BlockSpec((1, H1), lambda b: (0, 0)),
            pl.BlockSpec((H1, H2), lambda b: (0, 0)),
            pl.BlockSpec((1, H2), lambda b: (0, 0)),
            pl.BlockSpec((2 * H2, OUT), lambda b: (0, 0)),
            pl.BlockSpec((1, OUT), lambda b: (0, 0)),
        ],
        out_specs=pl.BlockSpec((1, 1, OUT),
                               lambda b: (jnp.maximum(b - 1, 0), 0, 0)),
        out_shape=jax.ShapeDtypeStruct((B, 1, OUT), jnp.float32),
        scratch_shapes=[
            pltpu.VMEM((2, N, N), jnp.bfloat16),
            pltpu.VMEM((2, N, H2), jnp.float32),
        ],
        compiler_params=pltpu.CompilerParams(
            vmem_limit_bytes=120 * 1024 * 1024,
        ),
    )(x, support, W1, b1_2d, W2, b2_2d, Wp, bp_2d)
    return out.reshape(B, OUT)


# speedup vs baseline: 1.3791x; 1.3791x over previous
import jax
import jax.numpy as jnp
from jax.experimental import pallas as pl
from jax.experimental.pallas import tpu as pltpu

_CHUNKS = 4


def _gcn_kernel(x_ref, s_ref, w1_ref, b1_ref, w2_ref, b2_ref, wp_ref,
                bp_ref, o_ref):
    n = s_ref.shape[1]
    rows = n // _CHUNKS

    t1 = jnp.dot(x_ref[0], w1_ref[...],
                 preferred_element_type=jnp.float32)
    t1b = t1.astype(jnp.bfloat16)

    sb_chunks = []
    t2_chunks = []
    for r in range(_CHUNKS):
        sc = s_ref[0, r * rows:(r + 1) * rows, :].astype(jnp.bfloat16)
        sb_chunks.append(sc)
        h1 = jnp.dot(sc, t1b, preferred_element_type=jnp.float32)
        h1 = jnp.maximum(h1 + b1_ref[...], 0.0)
        t2_chunks.append(jnp.dot(h1, w2_ref[...],
                                 preferred_element_type=jnp.float32))
    t2b = jnp.concatenate(t2_chunks, axis=0).astype(jnp.bfloat16)

    mx_parts = []
    sm_parts = []
    for r in range(_CHUNKS):
        h2 = jnp.dot(sb_chunks[r], t2b, preferred_element_type=jnp.float32)
        h2 = jnp.maximum(h2 + b2_ref[...], 0.0)
        mx_parts.append(jnp.max(h2, axis=0, keepdims=True))
        sm_parts.append(jnp.sum(h2, axis=0, keepdims=True))
    mx = jnp.max(jnp.concatenate(mx_parts, axis=0), axis=0, keepdims=True)
    sm = jnp.sum(jnp.concatenate(sm_parts, axis=0), axis=0, keepdims=True)

    cat = jnp.concatenate([mx, sm], axis=1)
    o_ref[0] = jnp.dot(cat, wp_ref[...],
                       preferred_element_type=jnp.float32) + bp_ref[...]


def kernel(x, support, W1, b1, W2, b2, Wp, bp):
    B, N, D_IN = x.shape
    H1 = W1.shape[1]
    H2 = W2.shape[1]
    OUT = Wp.shape[1]

    b1_2d = b1.reshape(1, H1)
    b2_2d = b2.reshape(1, H2)
    bp_2d = bp.reshape(1, OUT)

    out = pl.pallas_call(
        _gcn_kernel,
        grid=(B,),
        in_specs=[
            pl.BlockSpec((1, N, D_IN), lambda b: (b, 0, 0)),
            pl.BlockSpec((1, N, N), lambda b: (b, 0, 0)),
            pl.BlockSpec((D_IN, H1), lambda b: (0, 0)),
            pl.BlockSpec((1, H1), lambda b: (0, 0)),
            pl.BlockSpec((H1, H2), lambda b: (0, 0)),
            pl.BlockSpec((1, H2), lambda b: (0, 0)),
            pl.BlockSpec((2 * H2, OUT), lambda b: (0, 0)),
            pl.BlockSpec((1, OUT), lambda b: (0, 0)),
        ],
        out_specs=pl.BlockSpec((1, 1, OUT), lambda b: (b, 0, 0)),
        out_shape=jax.ShapeDtypeStruct((B, 1, OUT), jnp.float32),
        compiler_params=pltpu.CompilerParams(
            vmem_limit_bytes=100 * 1024 * 1024,
            dimension_semantics=("parallel",),
        ),
    )(x, support, W1, b1_2d, W2, b2_2d, Wp, bp_2d)
    return out.reshape(B, OUT)


# speedup vs baseline: 1.6411x; 1.1899x over previous
import jax
import jax.numpy as jnp
from jax.experimental import pallas as pl
from jax.experimental.pallas import tpu as pltpu

_CHUNKS = 8


def _gcn_kernel(x_ref, s_ref, w1_ref, b1_ref, w2_ref, b2_ref, wp_ref,
                bp_ref, o_ref):
    n = s_ref.shape[1]
    rows = n // _CHUNKS

    t1 = jnp.dot(x_ref[0], w1_ref[...],
                 preferred_element_type=jnp.float32)
    t1b = t1.astype(jnp.bfloat16)

    sb_chunks = []
    t2_chunks = []
    for r in range(_CHUNKS):
        sc = s_ref[0, r * rows:(r + 1) * rows, :].astype(jnp.bfloat16)
        sb_chunks.append(sc)
        h1 = jnp.dot(sc, t1b, preferred_element_type=jnp.float32)
        h1 = jnp.maximum(h1 + b1_ref[...], 0.0)
        t2_chunks.append(jnp.dot(h1, w2_ref[...],
                                 preferred_element_type=jnp.float32))
    t2b = jnp.concatenate(t2_chunks, axis=0).astype(jnp.bfloat16)

    mx_parts = []
    sm_parts = []
    for r in range(_CHUNKS):
        h2 = jnp.dot(sb_chunks[r], t2b, preferred_element_type=jnp.float32)
        h2 = jnp.maximum(h2 + b2_ref[...], 0.0)
        mx_parts.append(jnp.max(h2, axis=0, keepdims=True))
        sm_parts.append(jnp.sum(h2, axis=0, keepdims=True))
    mx = jnp.max(jnp.concatenate(mx_parts, axis=0), axis=0, keepdims=True)
    sm = jnp.sum(jnp.concatenate(sm_parts, axis=0), axis=0, keepdims=True)

    cat = jnp.concatenate([mx, sm], axis=1)
    o_ref[0] = jnp.dot(cat, wp_ref[...],
                       preferred_element_type=jnp.float32) + bp_ref[...]


def kernel(x, support, W1, b1, W2, b2, Wp, bp):
    B, N, D_IN = x.shape
    H1 = W1.shape[1]
    H2 = W2.shape[1]
    OUT = Wp.shape[1]

    b1_2d = b1.reshape(1, H1)
    b2_2d = b2.reshape(1, H2)
    bp_2d = bp.reshape(1, OUT)

    out = pl.pallas_call(
        _gcn_kernel,
        grid=(B,),
        in_specs=[
            pl.BlockSpec((1, N, D_IN), lambda b: (b, 0, 0)),
            pl.BlockSpec((1, N, N), lambda b: (b, 0, 0)),
            pl.BlockSpec((D_IN, H1), lambda b: (0, 0)),
            pl.BlockSpec((1, H1), lambda b: (0, 0)),
            pl.BlockSpec((H1, H2), lambda b: (0, 0)),
            pl.BlockSpec((1, H2), lambda b: (0, 0)),
            pl.BlockSpec((2 * H2, OUT), lambda b: (0, 0)),
            pl.BlockSpec((1, OUT), lambda b: (0, 0)),
        ],
        out_specs=pl.BlockSpec((1, 1, OUT), lambda b: (b, 0, 0)),
        out_shape=jax.ShapeDtypeStruct((B, 1, OUT), jnp.float32),
        compiler_params=pltpu.CompilerParams(
            vmem_limit_bytes=100 * 1024 * 1024,
            dimension_semantics=("parallel",),
        ),
    )(x, support, W1, b1_2d, W2, b2_2d, Wp, bp_2d)
    return out.reshape(B, OUT)
